# baseline (speedup 1.0000x reference)
import functools

import jax
import jax.numpy as jnp
from jax import lax
from jax.experimental import pallas as pl
from jax.experimental.pallas import tpu as pltpu
from jax.experimental.pallas import tpu_sc as plsc


def kernel(sessions, pos_emb):
    B, L, F = sessions.shape
    mesh = plsc.VectorSubcoreMesh(core_axis_name="c", subcore_axis_name="s")
    NW = mesh.num_cores * mesh.num_subcores
    SESS_PER_W = B // NW
    CHUNK = 4
    N_LANES = 16
    K = F // N_LANES

    @functools.partial(
        pl.kernel,
        out_type=jax.ShapeDtypeStruct((B, L, F), sessions.dtype),
        mesh=mesh,
        scratch_types=[
            pltpu.VMEM((L, F), jnp.float32),
            pltpu.VMEM((CHUNK, L, F), jnp.float32),
        ],
    )
    def sc_kernel(sess_hbm, pos_hbm, out_hbm, tbl_v, buf_v):
        wid = lax.axis_index("s") * mesh.num_cores + lax.axis_index("c")
        pltpu.sync_copy(pos_hbm, tbl_v)
        base = wid * SESS_PER_W

        def chunk_body(c, _):
            b0 = base + c * CHUNK
            pltpu.sync_copy(sess_hbm.at[pl.ds(b0, CHUNK)], buf_v)

            def row_body(t, _):
                tr = L - 1 - t
                for ci in range(CHUNK):
                    for k in range(K):
                        sl = pl.ds(k * N_LANES, N_LANES)
                        buf_v[ci, t, sl] = buf_v[ci, t, sl] + tbl_v[tr, sl]
                return _

            lax.fori_loop(0, L, row_body, None)
            pltpu.sync_copy(buf_v, out_hbm.at[pl.ds(b0, CHUNK)])
            return _

        lax.fori_loop(0, SESS_PER_W // CHUNK, chunk_body, None)

    return sc_kernel(sessions, pos_emb)


# speedup vs baseline: 4.4835x; 4.4835x over previous
import functools

import jax
import jax.numpy as jnp
from jax import lax
from jax.experimental import pallas as pl
from jax.experimental.pallas import tpu as pltpu
from jax.experimental.pallas import tpu_sc as plsc


def _flip_table_on_sc(pos_emb):
    L, F = pos_emb.shape
    mesh = plsc.ScalarSubcoreMesh(axis_name="c", num_cores=2)
    half = L // 2

    @functools.partial(
        pl.kernel,
        out_type=jax.ShapeDtypeStruct((L, F), pos_emb.dtype),
        mesh=mesh,
        scratch_types=[pltpu.SemaphoreType.DMA],
    )
    def flip_kernel(pos_hbm, out_hbm, sem):
        cid = lax.axis_index("c")
        base = cid * half
        handles = [
            pltpu.async_copy(
                pos_hbm.at[pl.ds(L - 1 - base - i, 1)],
                out_hbm.at[pl.ds(base + i, 1)],
                sem,
            )
            for i in range(half)
        ]
        for h in handles:
            h.wait()

    return flip_kernel(pos_emb)


def _add_body(s_ref, pf_ref, o_ref):
    o_ref[...] = s_ref[...] + pf_ref[...][None, :, :]


def kernel(sessions, pos_emb):
    B, L, F = sessions.shape
    flipped = _flip_table_on_sc(pos_emb)
    BLK_B = 128
    return pl.pallas_call(
        _add_body,
        grid=(B // BLK_B,),
        in_specs=[
            pl.BlockSpec((BLK_B, L, F), lambda i: (i, 0, 0)),
            pl.BlockSpec((L, F), lambda i: (0, 0)),
        ],
        out_specs=pl.BlockSpec((BLK_B, L, F), lambda i: (i, 0, 0)),
        out_shape=jax.ShapeDtypeStruct((B, L, F), sessions.dtype),
    )(sessions, flipped)


# speedup vs baseline: 4.5159x; 1.0072x over previous
import functools

import jax
import jax.numpy as jnp
from jax import lax
from jax.experimental import pallas as pl
from jax.experimental.pallas import tpu as pltpu
from jax.experimental.pallas import tpu_sc as plsc


_ROWS_PER_WORKER = 8


def _flip_table_on_sc(pos_emb):
    L, F = pos_emb.shape
    n_active = L // _ROWS_PER_WORKER
    mesh = plsc.VectorSubcoreMesh(core_axis_name="c", subcore_axis_name="s")

    @functools.partial(
        pl.kernel,
        out_type=jax.ShapeDtypeStruct((L, F), pos_emb.dtype),
        mesh=mesh,
        scratch_types=[
            pltpu.VMEM((_ROWS_PER_WORKER, F), pos_emb.dtype),
            pltpu.SemaphoreType.DMA,
        ],
    )
    def flip_kernel(pos_hbm, out_hbm, buf_v, sem):
        wid = lax.axis_index("s") * mesh.num_cores + lax.axis_index("c")

        @pl.when(wid < n_active)
        def _():
            base = wid * _ROWS_PER_WORKER
            handles = [
                pltpu.async_copy(
                    pos_hbm.at[pl.ds(L - 1 - base - i, 1)],
                    buf_v.at[pl.ds(i, 1)],
                    sem,
                )
                for i in range(_ROWS_PER_WORKER)
            ]
            for h in handles:
                h.wait()
            pltpu.sync_copy(buf_v, out_hbm.at[pl.ds(base, _ROWS_PER_WORKER)])

    return flip_kernel(pos_emb)


def _add_body(s_ref, pf_ref, o_ref):
    o_ref[...] = s_ref[...] + pf_ref[...][None, :, :]


def kernel(sessions, pos_emb):
    B, L, F = sessions.shape
    flipped = _flip_table_on_sc(pos_emb)
    BLK_B = 128
    return pl.pallas_call(
        _add_body,
        grid=(B // BLK_B,),
        in_specs=[
            pl.BlockSpec((BLK_B, L, F), lambda i: (i, 0, 0)),
            pl.BlockSpec((L, F), lambda i: (0, 0)),
        ],
        out_specs=pl.BlockSpec((BLK_B, L, F), lambda i: (i, 0, 0)),
        out_shape=jax.ShapeDtypeStruct((B, L, F), sessions.dtype),
    )(sessions, flipped)
